# baseline (speedup 1.0000x reference)
import jax
import jax.numpy as jnp
from jax.experimental import pallas as pl
from jax.experimental.pallas import tpu as pltpu

NPTS = 4096
N = 16384
B = 8
LANES = 128
NGROUPS = 8


def _fps_kernel(x_ref, y_ref, z_ref, idx_ref, ox_ref, oy_ref, oz_ref,
                dists_ref):
    nchunks = x_ref.shape[1] // LANES
    lane_iota_f = jax.lax.broadcasted_iota(
        jnp.int32, (B, LANES), 1).astype(jnp.float32)

    dists_ref[...] = jnp.full((B, N), 1e10, dtype=jnp.float32)
    cx0, cy0, cz0 = x_ref[:, 0:1], y_ref[:, 0:1], z_ref[:, 0:1]
    zero_lane = lane_iota_f == 0.0
    acc0 = (jnp.zeros((B, LANES), jnp.int32),
            jnp.where(zero_lane, cx0, 0.0),
            jnp.where(zero_lane, cy0, 0.0),
            jnp.where(zero_lane, cz0, 0.0))
    c0 = (cx0, cy0, cz0) + acc0

    def body(i, carry):
        cx, cy, cz, aidx, ax, ay, az = carry
        champs = []
        per_group = nchunks // NGROUPS
        for g in range(NGROUPS):
            champ = None
            for kk in range(per_group):
                k = g * per_group + kk
                sl = pl.ds(k * LANES, LANES)
                xk = x_ref[:, sl]
                yk = y_ref[:, sl]
                zk = z_ref[:, sl]
                dx = xk - cx
                dy = yk - cy
                dz = zk - cz
                d = (dx * dx + dz * dz) + dy * dy
                v = jnp.minimum(dists_ref[:, sl], d)
                dists_ref[:, sl] = v
                node = (v, lane_iota_f + jnp.float32(k * LANES), xk, yk, zk)
                if champ is None:
                    champ = node
                else:
                    keep = champ[0] >= node[0]
                    champ = tuple(
                        jnp.where(keep, fa, fb)
                        for fa, fb in zip(champ, node))
            champs.append(champ)
        while len(champs) > 1:
            nxt_champs = []
            for k in range(0, len(champs), 2):
                a, b = champs[k], champs[k + 1]
                take_a = a[0] >= b[0]
                nxt_champs.append(tuple(
                    jnp.where(take_a, fa, fb) for fa, fb in zip(a, b)))
            champs = nxt_champs
        v, iw, xw, yw, zw = champs[0]
        m = jnp.max(v, axis=1, keepdims=True)
        nxt_f = jnp.min(
            jnp.where(v == m, iw, jnp.float32(N)), axis=1, keepdims=True
        )
        onehot = iw == nxt_f
        nxt = nxt_f.astype(jnp.int32)
        ncx = jnp.max(jnp.where(onehot, xw, -jnp.inf), axis=1, keepdims=True)
        ncy = jnp.max(jnp.where(onehot, yw, -jnp.inf), axis=1, keepdims=True)
        ncz = jnp.max(jnp.where(onehot, zw, -jnp.inf), axis=1, keepdims=True)
        lane_sel = lane_iota_f == (i % LANES).astype(jnp.float32)
        aidx = jnp.where(lane_sel, nxt, aidx)
        ax = jnp.where(lane_sel, ncx, ax)
        ay = jnp.where(lane_sel, ncy, ay)
        az = jnp.where(lane_sel, ncz, az)

        @pl.when(i % LANES == LANES - 1)
        def _flush():
            base = pl.multiple_of(i - (LANES - 1), LANES)
            idx_ref[pl.ds(base, LANES), :] = aidx.T
            ox_ref[pl.ds(base, LANES), :] = ax.T
            oy_ref[pl.ds(base, LANES), :] = ay.T
            oz_ref[pl.ds(base, LANES), :] = az.T

        return (ncx, ncy, ncz, aidx, ax, ay, az)

    jax.lax.fori_loop(1, idx_ref.shape[0], body, c0)


@jax.jit
def kernel(inp):
    x = inp[:, :, 0]
    y = inp[:, :, 1]
    z = inp[:, :, 2]
    out_types = (
        jax.ShapeDtypeStruct((NPTS, B), jnp.int32),
        jax.ShapeDtypeStruct((NPTS, B), jnp.float32),
        jax.ShapeDtypeStruct((NPTS, B), jnp.float32),
        jax.ShapeDtypeStruct((NPTS, B), jnp.float32),
    )
    idx_t, ox_t, oy_t, oz_t = pl.pallas_call(
        _fps_kernel,
        out_shape=out_types,
        in_specs=[
            pl.BlockSpec((B, N), lambda: (0, 0)),
            pl.BlockSpec((B, N), lambda: (0, 0)),
            pl.BlockSpec((B, N), lambda: (0, 0)),
        ],
        out_specs=(
            pl.BlockSpec((NPTS, B), lambda: (0, 0)),
            pl.BlockSpec((NPTS, B), lambda: (0, 0)),
            pl.BlockSpec((NPTS, B), lambda: (0, 0)),
            pl.BlockSpec((NPTS, B), lambda: (0, 0)),
        ),
        scratch_shapes=[pltpu.VMEM((B, N), jnp.float32)],
    )(x, y, z)
    idx = idx_t.T
    out = jnp.stack([ox_t.T, oy_t.T, oz_t.T], axis=-1)
    return (out, idx)


# speedup vs baseline: 1.1273x; 1.1273x over previous
import jax
import jax.numpy as jnp
from jax.experimental import pallas as pl
from jax.experimental.pallas import tpu as pltpu

NPTS = 4096
N = 16384
B = 8
LANES = 128
NGROUPS = 2


def _fps_kernel(x_ref, y_ref, z_ref, idx_ref, ox_ref, oy_ref, oz_ref,
                dists_ref):
    nchunks = x_ref.shape[1] // LANES
    lane_iota_f = jax.lax.broadcasted_iota(
        jnp.int32, (B, LANES), 1).astype(jnp.float32)

    dists_ref[...] = jnp.full((B, N), 1e10, dtype=jnp.float32)
    cx0, cy0, cz0 = x_ref[:, 0:1], y_ref[:, 0:1], z_ref[:, 0:1]
    zero_lane = lane_iota_f == 0.0
    acc0 = (jnp.zeros((B, LANES), jnp.int32),
            jnp.where(zero_lane, cx0, 0.0),
            jnp.where(zero_lane, cy0, 0.0),
            jnp.where(zero_lane, cz0, 0.0))
    c0 = (cx0, cy0, cz0) + acc0

    def body(i, carry):
        cx, cy, cz, aidx, ax, ay, az = carry
        champs = []
        per_group = nchunks // NGROUPS
        for g in range(NGROUPS):
            champ = None
            for kk in range(per_group):
                k = g * per_group + kk
                sl = pl.ds(k * LANES, LANES)
                xk = x_ref[:, sl]
                yk = y_ref[:, sl]
                zk = z_ref[:, sl]
                dx = xk - cx
                dy = yk - cy
                dz = zk - cz
                d = (dx * dx + dz * dz) + dy * dy
                v = jnp.minimum(dists_ref[:, sl], d)
                dists_ref[:, sl] = v
                node = (v, lane_iota_f + jnp.float32(k * LANES), xk, yk, zk)
                if champ is None:
                    champ = node
                else:
                    keep = champ[0] >= node[0]
                    champ = tuple(
                        jnp.where(keep, fa, fb)
                        for fa, fb in zip(champ, node))
            champs.append(champ)
        while len(champs) > 1:
            nxt_champs = []
            for k in range(0, len(champs), 2):
                a, b = champs[k], champs[k + 1]
                take_a = a[0] >= b[0]
                nxt_champs.append(tuple(
                    jnp.where(take_a, fa, fb) for fa, fb in zip(a, b)))
            champs = nxt_champs
        v, iw, xw, yw, zw = champs[0]
        m = jnp.max(v, axis=1, keepdims=True)
        nxt_f = jnp.min(
            jnp.where(v == m, iw, jnp.float32(N)), axis=1, keepdims=True
        )
        onehot = iw == nxt_f
        nxt = nxt_f.astype(jnp.int32)
        ncx = jnp.max(jnp.where(onehot, xw, -jnp.inf), axis=1, keepdims=True)
        ncy = jnp.max(jnp.where(onehot, yw, -jnp.inf), axis=1, keepdims=True)
        ncz = jnp.max(jnp.where(onehot, zw, -jnp.inf), axis=1, keepdims=True)
        lane_sel = lane_iota_f == (i % LANES).astype(jnp.float32)
        aidx = jnp.where(lane_sel, nxt, aidx)
        ax = jnp.where(lane_sel, ncx, ax)
        ay = jnp.where(lane_sel, ncy, ay)
        az = jnp.where(lane_sel, ncz, az)

        @pl.when(i % LANES == LANES - 1)
        def _flush():
            base = pl.multiple_of(i - (LANES - 1), LANES)
            idx_ref[pl.ds(base, LANES), :] = aidx.T
            ox_ref[pl.ds(base, LANES), :] = ax.T
            oy_ref[pl.ds(base, LANES), :] = ay.T
            oz_ref[pl.ds(base, LANES), :] = az.T

        return (ncx, ncy, ncz, aidx, ax, ay, az)

    jax.lax.fori_loop(1, idx_ref.shape[0], body, c0)


@jax.jit
def kernel(inp):
    x = inp[:, :, 0]
    y = inp[:, :, 1]
    z = inp[:, :, 2]
    out_types = (
        jax.ShapeDtypeStruct((NPTS, B), jnp.int32),
        jax.ShapeDtypeStruct((NPTS, B), jnp.float32),
        jax.ShapeDtypeStruct((NPTS, B), jnp.float32),
        jax.ShapeDtypeStruct((NPTS, B), jnp.float32),
    )
    idx_t, ox_t, oy_t, oz_t = pl.pallas_call(
        _fps_kernel,
        out_shape=out_types,
        in_specs=[
            pl.BlockSpec((B, N), lambda: (0, 0)),
            pl.BlockSpec((B, N), lambda: (0, 0)),
            pl.BlockSpec((B, N), lambda: (0, 0)),
        ],
        out_specs=(
            pl.BlockSpec((NPTS, B), lambda: (0, 0)),
            pl.BlockSpec((NPTS, B), lambda: (0, 0)),
            pl.BlockSpec((NPTS, B), lambda: (0, 0)),
            pl.BlockSpec((NPTS, B), lambda: (0, 0)),
        ),
        scratch_shapes=[pltpu.VMEM((B, N), jnp.float32)],
    )(x, y, z)
    idx = idx_t.T
    out = jnp.stack([ox_t.T, oy_t.T, oz_t.T], axis=-1)
    return (out, idx)
